# Initial kernel scaffold; baseline (speedup 1.0000x reference)
import functools

import jax
import jax.numpy as jnp
from jax import lax
from jax.experimental import pallas as pl
from jax.experimental.pallas import tpu as pltpu
from jax.experimental.pallas import tpu_sc as plsc

N = 10000
E = 320000
D = 128

NC = 2
NS = 16
NW = NC * NS
EPW = E // NW
CHUNK = 125
ITERS = EPW // CHUNK
RPT = N // NS
CW = 16

_mesh = plsc.VectorSubcoreMesh(core_axis_name="c", subcore_axis_name="s")


@functools.partial(
    pl.kernel,
    out_type=[
        jax.ShapeDtypeStruct((NC, NS, RPT, D), jnp.float32),
        jax.ShapeDtypeStruct((NC, NS, RPT, CW), jnp.float32),
    ],
    mesh=_mesh,
    scratch_types=[
        pltpu.VMEM((ITERS, CHUNK), jnp.int32),
        pltpu.VMEM((ITERS, CHUNK), jnp.int32),
        pltpu.VMEM((CHUNK, D), jnp.float32),
        pltpu.VMEM((CHUNK, CW), jnp.float32),
        pltpu.VMEM_SHARED((N, D), jnp.float32),
        pltpu.VMEM_SHARED((N, CW), jnp.float32),
        pltpu.SemaphoreType.DMA,
    ],
)
def _sc_aggregate(src_hbm, dst_hbm, x_hbm, zrow_hbm, zcnt_hbm, ones_hbm,
                  out_sum, out_cnt,
                  src_v, dst_v, rows_v, ones_v, acc_sh, cnt_sh, sem):
    c = lax.axis_index("c")
    s = lax.axis_index("s")
    wid = s * NC + c

    pltpu.sync_copy(zrow_hbm, acc_sh.at[pl.ds(s * RPT, RPT)])
    pltpu.sync_copy(zcnt_hbm, cnt_sh.at[pl.ds(s * RPT, RPT)])
    pltpu.sync_copy(ones_hbm, ones_v)
    pltpu.sync_copy(src_hbm.at[wid], src_v)
    pltpu.sync_copy(dst_hbm.at[wid], dst_v)
    plsc.subcore_barrier()

    def body(j, carry):
        pltpu.async_copy(x_hbm.at[src_v.at[j]], rows_v, sem).wait()
        pltpu.sync_copy(rows_v, acc_sh.at[dst_v.at[j]], add=True)
        pltpu.sync_copy(ones_v, cnt_sh.at[dst_v.at[j]], add=True)
        return carry

    lax.fori_loop(0, ITERS, body, 0)
    plsc.subcore_barrier()

    pltpu.sync_copy(acc_sh.at[pl.ds(s * RPT, RPT)], out_sum.at[c, s])
    pltpu.sync_copy(cnt_sh.at[pl.ds(s * RPT, RPT)], out_cnt.at[c, s])


_BR = 1000
_NB = N // _BR


def _tc_dense(psum_ref, pcnt_ref, x_ref, wl_ref, wr_ref, b_ref,
              h_ref, st_ref):
    summed = psum_ref[0] + psum_ref[1]
    cnt = pcnt_ref[0, :, 0:1] + pcnt_ref[1, :, 0:1]
    mean = summed / jnp.maximum(cnt, 1.0)
    h = jnp.dot(mean, wl_ref[...], preferred_element_type=jnp.float32)
    h = h + jnp.dot(x_ref[...], wr_ref[...], preferred_element_type=jnp.float32)
    h = jnp.maximum(h + b_ref[...], 0.0)
    h_ref[...] = h
    st_ref[...] = jnp.concatenate(
        [jnp.sum(h, axis=0, keepdims=True),
         jnp.sum(h * h, axis=0, keepdims=True)], axis=0)[None]


def _tc_norm(h_ref, st_ref, g_ref, be_ref, o_ref):
    st = st_ref[...]
    mu = jnp.sum(st[:, 0, :], axis=0, keepdims=True) * (1.0 / N)
    ex2 = jnp.sum(st[:, 1, :], axis=0, keepdims=True) * (1.0 / N)
    var = ex2 - mu * mu
    scale = g_ref[...] * lax.rsqrt(var + 1e-5)
    o_ref[...] = (h_ref[...] - mu) * scale + be_ref[...]


def kernel(x, edge_index, W_l, b_l, W_r, gamma, beta):
    src = edge_index[0].reshape(NW, ITERS, CHUNK)
    dst = edge_index[1].reshape(NW, ITERS, CHUNK)
    zrow = jnp.zeros((RPT, D), jnp.float32)
    zcnt = jnp.zeros((RPT, CW), jnp.float32)
    ones = jnp.ones((CHUNK, CW), jnp.float32)

    psum, pcnt = _sc_aggregate(src, dst, x, zrow, zcnt, ones)
    psum = psum.reshape(NC, N, D)
    pcnt = pcnt.reshape(NC, N, CW)

    h, stats = pl.pallas_call(
        _tc_dense,
        grid=(_NB,),
        in_specs=[
            pl.BlockSpec((NC, _BR, D), lambda i: (0, i, 0)),
            pl.BlockSpec((NC, _BR, CW), lambda i: (0, i, 0)),
            pl.BlockSpec((_BR, D), lambda i: (i, 0)),
            pl.BlockSpec((D, D), lambda i: (0, 0)),
            pl.BlockSpec((D, D), lambda i: (0, 0)),
            pl.BlockSpec((1, D), lambda i: (0, 0)),
        ],
        out_specs=[
            pl.BlockSpec((_BR, D), lambda i: (i, 0)),
            pl.BlockSpec((1, 2, D), lambda i: (i, 0, 0)),
        ],
        out_shape=[
            jax.ShapeDtypeStruct((N, D), jnp.float32),
            jax.ShapeDtypeStruct((_NB, 2, D), jnp.float32),
        ],
    )(psum, pcnt, x, W_l.T, W_r.T, b_l.reshape(1, D))

    out = pl.pallas_call(
        _tc_norm,
        grid=(_NB,),
        in_specs=[
            pl.BlockSpec((_BR, D), lambda i: (i, 0)),
            pl.BlockSpec((_NB, 2, D), lambda i: (0, 0, 0)),
            pl.BlockSpec((1, D), lambda i: (0, 0)),
            pl.BlockSpec((1, D), lambda i: (0, 0)),
        ],
        out_specs=pl.BlockSpec((_BR, D), lambda i: (i, 0)),
        out_shape=jax.ShapeDtypeStruct((N, D), jnp.float32),
    )(h, stats, gamma.reshape(1, D), beta.reshape(1, D))
    return out


# speedup vs baseline: 4.5266x; 4.5266x over previous
import functools

import jax
import jax.numpy as jnp
from jax import lax
from jax.experimental import pallas as pl
from jax.experimental.pallas import tpu as pltpu
from jax.experimental.pallas import tpu_sc as plsc

N = 10000
E = 320000
D = 128

NC = 2
NS = 16
NW = NC * NS
EPW = E // NW
CHUNK = 80
ITERS = EPW // CHUNK
VR = 16
NVR = CHUNK // VR
NP_ = 10240
RPT = NP_ // NS
STG = CHUNK
NHOP = RPT // STG

_mesh = plsc.VectorSubcoreMesh(core_axis_name="c", subcore_axis_name="s")


@functools.partial(
    pl.kernel,
    out_type=[
        jax.ShapeDtypeStruct((NC, NS, RPT, D), jnp.float32),
        jax.ShapeDtypeStruct((NC, NS, RPT, D), jnp.float32),
    ],
    mesh=_mesh,
    scratch_types=[
        pltpu.VMEM((CHUNK,), jnp.int32),
        pltpu.VMEM((CHUNK,), jnp.int32),
        pltpu.VMEM((CHUNK, D), jnp.float32),
        pltpu.VMEM((CHUNK, D), jnp.float32),
        pltpu.VMEM_SHARED((NP_, D), jnp.float32),
        pltpu.SemaphoreType.DMA,
    ],
)
def _sc_aggregate(src_hbm, dst_hbm, x_hbm, zrow_hbm, ones_hbm,
                  out_sum, out_cnt,
                  src_v, dst_v, rows_v, ones_v, acc_sh, sem):
    c = lax.axis_index("c")
    s = lax.axis_index("s")
    wid = s * NC + c

    pltpu.sync_copy(zrow_hbm, rows_v)
    pltpu.sync_copy(ones_hbm, ones_v)
    for k in range(NHOP):
        pltpu.sync_copy(rows_v, acc_sh.at[pl.ds(s * RPT + k * STG, STG)])
    plsc.subcore_barrier()

    def body(j, carry):
        pltpu.sync_copy(src_hbm.at[wid, j], src_v)
        pltpu.sync_copy(dst_hbm.at[wid, j], dst_v)
        pltpu.async_copy(x_hbm.at[src_v], rows_v, sem).wait()
        pltpu.sync_copy(rows_v, acc_sh.at[dst_v], add=True)
        return carry

    lax.fori_loop(0, ITERS, body, 0)
    plsc.subcore_barrier()

    pltpu.sync_copy(zrow_hbm, ones_v)
    for k in range(NHOP):
        pltpu.sync_copy(acc_sh.at[pl.ds(s * RPT + k * STG, STG)], rows_v)
        pltpu.sync_copy(rows_v, out_sum.at[c, s, pl.ds(k * STG, STG)])
        pltpu.sync_copy(ones_v, acc_sh.at[pl.ds(s * RPT + k * STG, STG)])
    pltpu.sync_copy(ones_hbm, ones_v)
    plsc.subcore_barrier()

    def body_cnt(j, carry):
        pltpu.sync_copy(dst_hbm.at[wid, j], dst_v)
        pltpu.sync_copy(ones_v, acc_sh.at[dst_v], add=True)
        return carry

    lax.fori_loop(0, ITERS, body_cnt, 0)
    plsc.subcore_barrier()

    for k in range(NHOP):
        pltpu.sync_copy(acc_sh.at[pl.ds(s * RPT + k * STG, STG)], rows_v)
        pltpu.sync_copy(rows_v, out_cnt.at[c, s, pl.ds(k * STG, STG)])


_BR = 1024
_NB = NP_ // _BR


def _tc_dense(psum_ref, pcnt_ref, x_ref, wl_ref, wr_ref, b_ref,
              h_ref, st_ref):
    summed = psum_ref[0] + psum_ref[1]
    cnt = pcnt_ref[0, :, 0:1] + pcnt_ref[1, :, 0:1]
    mean = summed / jnp.maximum(cnt, 1.0)
    h = jnp.dot(mean, wl_ref[...], preferred_element_type=jnp.float32)
    h = h + jnp.dot(x_ref[...], wr_ref[...], preferred_element_type=jnp.float32)
    h = jnp.maximum(h + b_ref[...], 0.0)
    h_ref[...] = h
    rows = lax.broadcasted_iota(jnp.int32, (_BR, 1), 0) + pl.program_id(0) * _BR
    hm = jnp.where(rows < N, h, 0.0)
    st_ref[...] = jnp.concatenate(
        [jnp.sum(hm, axis=0, keepdims=True),
         jnp.sum(hm * hm, axis=0, keepdims=True)], axis=0)[None]


def _tc_norm(h_ref, st_ref, g_ref, be_ref, o_ref):
    st = st_ref[...]
    mu = jnp.sum(st[:, 0, :], axis=0, keepdims=True) * (1.0 / N)
    ex2 = jnp.sum(st[:, 1, :], axis=0, keepdims=True) * (1.0 / N)
    var = ex2 - mu * mu
    scale = g_ref[...] * lax.rsqrt(var + 1e-5)
    o_ref[...] = (h_ref[...] - mu) * scale + be_ref[...]


def kernel(x, edge_index, W_l, b_l, W_r, gamma, beta):
    src = edge_index[0].reshape(NW, ITERS, CHUNK)
    dst = edge_index[1].reshape(NW, ITERS, CHUNK)
    zrow = jnp.zeros((STG, D), jnp.float32)
    ones = jnp.ones((CHUNK, D), jnp.float32)

    psum, pcnt = _sc_aggregate(src, dst, x, zrow, ones)
    psum = psum.reshape(NC, NP_, D)
    pcnt = pcnt.reshape(NC, NP_, D)

    h, stats = pl.pallas_call(
        _tc_dense,
        grid=(_NB,),
        in_specs=[
            pl.BlockSpec((NC, _BR, D), lambda i: (0, i, 0)),
            pl.BlockSpec((NC, _BR, D), lambda i: (0, i, 0)),
            pl.BlockSpec((_BR, D), lambda i: (i, 0)),
            pl.BlockSpec((D, D), lambda i: (0, 0)),
            pl.BlockSpec((D, D), lambda i: (0, 0)),
            pl.BlockSpec((1, D), lambda i: (0, 0)),
        ],
        out_specs=[
            pl.BlockSpec((_BR, D), lambda i: (i, 0)),
            pl.BlockSpec((1, 2, D), lambda i: (i, 0, 0)),
        ],
        out_shape=[
            jax.ShapeDtypeStruct((N, D), jnp.float32),
            jax.ShapeDtypeStruct((_NB, 2, D), jnp.float32),
        ],
    )(psum, pcnt, x, W_l.T, W_r.T, b_l.reshape(1, D))

    out = pl.pallas_call(
        _tc_norm,
        grid=(_NB,),
        in_specs=[
            pl.BlockSpec((_BR, D), lambda i: (i, 0)),
            pl.BlockSpec((_NB, 2, D), lambda i: (0, 0, 0)),
            pl.BlockSpec((1, D), lambda i: (0, 0)),
            pl.BlockSpec((1, D), lambda i: (0, 0)),
        ],
        out_specs=pl.BlockSpec((_BR, D), lambda i: (i, 0)),
        out_shape=jax.ShapeDtypeStruct((N, D), jnp.float32),
    )(h, stats, gamma.reshape(1, D), beta.reshape(1, D))
    return out


# speedup vs baseline: 8.4421x; 1.8650x over previous
import functools

import jax
import jax.numpy as jnp
from jax import lax
from jax.experimental import pallas as pl
from jax.experimental.pallas import tpu as pltpu
from jax.experimental.pallas import tpu_sc as plsc

N = 10000
E = 320000
D = 128

NC = 2
NS = 16
NW = NC * NS
CH = 128
NCHUNK = E // CH
NP_ = 10240
RPT = NP_ // NS
STG = 128
NHOP = RPT // STG

_mesh = plsc.VectorSubcoreMesh(core_axis_name="c", subcore_axis_name="s")


@functools.partial(
    pl.kernel,
    out_type=[
        jax.ShapeDtypeStruct((NC, NS, RPT, D), jnp.float32),
        jax.ShapeDtypeStruct((NC, NS, RPT, D), jnp.float32),
    ],
    mesh=_mesh,
    scratch_types=[
        pltpu.VMEM((2, 2, CH), jnp.int32),
        pltpu.VMEM((2, CH, D), jnp.float32),
        pltpu.VMEM_SHARED((NP_, D), jnp.float32),
        pltpu.SemaphoreType.DMA,
        pltpu.SemaphoreType.DMA,
        pltpu.SemaphoreType.DMA,
        pltpu.SemaphoreType.DMA,
    ],
)
def _sc_aggregate(ei_hbm, x_hbm, zrow_hbm, ones_hbm,
                  out_sum, out_cnt,
                  idx_v, rows_v, acc_sh, sg0, sg1, ss0, ss1):
    c = lax.axis_index("c")
    s = lax.axis_index("s")
    wid = s * NC + c
    nt = jnp.where(wid < 4, (NCHUNK // NW) + 1, NCHUNK // NW)

    pltpu.sync_copy(zrow_hbm, rows_v.at[0])
    for k in range(NHOP):
        pltpu.sync_copy(rows_v.at[0], acc_sh.at[pl.ds(s * RPT + k * STG, STG)])
    plsc.subcore_barrier()

    def _gather_start(t, b):
        pltpu.sync_copy(ei_hbm.at[wid + t * NW], idx_v.at[b])

        @pl.when(b == 0)
        def _():
            pltpu.async_copy(x_hbm.at[idx_v.at[0, 0]], rows_v.at[0], sg0)

        @pl.when(b == 1)
        def _():
            pltpu.async_copy(x_hbm.at[idx_v.at[1, 0]], rows_v.at[1], sg1)

    def _gather_wait(b):
        @pl.when(b == 0)
        def _():
            pltpu.make_async_copy(x_hbm.at[idx_v.at[0, 0]], rows_v.at[0],
                                  sg0).wait()

        @pl.when(b == 1)
        def _():
            pltpu.make_async_copy(x_hbm.at[idx_v.at[1, 0]], rows_v.at[1],
                                  sg1).wait()

    _gather_start(0, 0)

    def body(t, carry):
        b = t & 1

        @pl.when(t + 1 < nt)
        def _():
            _gather_start(t + 1, 1 - b)

        _gather_wait(b)
        pltpu.sync_copy(rows_v.at[b], acc_sh.at[idx_v.at[b, 1]], add=True)
        return carry

    lax.fori_loop(0, nt, body, 0)
    plsc.subcore_barrier()

    pltpu.sync_copy(zrow_hbm, rows_v.at[1])
    for k in range(NHOP):
        pltpu.sync_copy(acc_sh.at[pl.ds(s * RPT + k * STG, STG)], rows_v.at[0])
        pltpu.sync_copy(rows_v.at[0], out_sum.at[c, s, pl.ds(k * STG, STG)])
        pltpu.sync_copy(rows_v.at[1], acc_sh.at[pl.ds(s * RPT + k * STG, STG)])
    plsc.subcore_barrier()

    pltpu.sync_copy(ones_hbm, rows_v.at[0])
    pltpu.sync_copy(ei_hbm.at[wid], idx_v.at[0])

    def _cnt_scatter_start(b):
        @pl.when(b == 0)
        def _():
            pltpu.async_copy(rows_v.at[0], acc_sh.at[idx_v.at[0, 1]], ss0,
                             add=True)

        @pl.when(b == 1)
        def _():
            pltpu.async_copy(rows_v.at[0], acc_sh.at[idx_v.at[1, 1]], ss1,
                             add=True)

    def _cnt_scatter_wait(b):
        @pl.when(b == 0)
        def _():
            pltpu.make_async_copy(rows_v.at[0], acc_sh.at[idx_v.at[0, 1]],
                                  ss0).wait()

        @pl.when(b == 1)
        def _():
            pltpu.make_async_copy(rows_v.at[0], acc_sh.at[idx_v.at[1, 1]],
                                  ss1).wait()

    def body_cnt(t, carry):
        b = t & 1

        @pl.when(t >= 1)
        def _():
            _cnt_scatter_wait(1 - b)

        @pl.when(t + 1 < nt)
        def _():
            pltpu.sync_copy(ei_hbm.at[wid + (t + 1) * NW], idx_v.at[1 - b])

        _cnt_scatter_start(b)
        return carry

    lax.fori_loop(0, nt, body_cnt, 0)
    _cnt_scatter_wait((nt - 1) & 1)
    plsc.subcore_barrier()

    for k in range(NHOP):
        pltpu.sync_copy(acc_sh.at[pl.ds(s * RPT + k * STG, STG)], rows_v.at[0])
        pltpu.sync_copy(rows_v.at[0], out_cnt.at[c, s, pl.ds(k * STG, STG)])


_BR = 1024
_NB = NP_ // _BR


def _tc_dense(psum_ref, pcnt_ref, x_ref, wl_ref, wr_ref, b_ref,
              h_ref, st_ref):
    summed = psum_ref[0] + psum_ref[1]
    cnt = pcnt_ref[0, :, 0:1] + pcnt_ref[1, :, 0:1]
    mean = summed / jnp.maximum(cnt, 1.0)
    h = jnp.dot(mean, wl_ref[...], preferred_element_type=jnp.float32)
    h = h + jnp.dot(x_ref[...], wr_ref[...], preferred_element_type=jnp.float32)
    h = jnp.maximum(h + b_ref[...], 0.0)
    h_ref[...] = h
    rows = lax.broadcasted_iota(jnp.int32, (_BR, 1), 0) + pl.program_id(0) * _BR
    hm = jnp.where(rows < N, h, 0.0)
    st_ref[...] = jnp.concatenate(
        [jnp.sum(hm, axis=0, keepdims=True),
         jnp.sum(hm * hm, axis=0, keepdims=True)], axis=0)[None]


def _tc_norm(h_ref, st_ref, g_ref, be_ref, o_ref):
    st = st_ref[...]
    mu = jnp.sum(st[:, 0, :], axis=0, keepdims=True) * (1.0 / N)
    ex2 = jnp.sum(st[:, 1, :], axis=0, keepdims=True) * (1.0 / N)
    var = ex2 - mu * mu
    scale = g_ref[...] * lax.rsqrt(var + 1e-5)
    o_ref[...] = (h_ref[...] - mu) * scale + be_ref[...]


def kernel(x, edge_index, W_l, b_l, W_r, gamma, beta):
    ei = jnp.transpose(edge_index.reshape(2, NCHUNK, CH), (1, 0, 2))
    zrow = jnp.zeros((STG, D), jnp.float32)
    ones = jnp.ones((CH, D), jnp.float32)

    psum, pcnt = _sc_aggregate(ei, x, zrow, ones)
    psum = psum.reshape(NC, NP_, D)
    pcnt = pcnt.reshape(NC, NP_, D)

    h, stats = pl.pallas_call(
        _tc_dense,
        grid=(_NB,),
        in_specs=[
            pl.BlockSpec((NC, _BR, D), lambda i: (0, i, 0)),
            pl.BlockSpec((NC, _BR, D), lambda i: (0, i, 0)),
            pl.BlockSpec((_BR, D), lambda i: (i, 0)),
            pl.BlockSpec((D, D), lambda i: (0, 0)),
            pl.BlockSpec((D, D), lambda i: (0, 0)),
            pl.BlockSpec((1, D), lambda i: (0, 0)),
        ],
        out_specs=[
            pl.BlockSpec((_BR, D), lambda i: (i, 0)),
            pl.BlockSpec((1, 2, D), lambda i: (i, 0, 0)),
        ],
        out_shape=[
            jax.ShapeDtypeStruct((N, D), jnp.float32),
            jax.ShapeDtypeStruct((_NB, 2, D), jnp.float32),
        ],
    )(psum, pcnt, x, W_l.T, W_r.T, b_l.reshape(1, D))

    out = pl.pallas_call(
        _tc_norm,
        grid=(_NB,),
        in_specs=[
            pl.BlockSpec((_BR, D), lambda i: (i, 0)),
            pl.BlockSpec((_NB, 2, D), lambda i: (0, 0, 0)),
            pl.BlockSpec((1, D), lambda i: (0, 0)),
            pl.BlockSpec((1, D), lambda i: (0, 0)),
        ],
        out_specs=pl.BlockSpec((_BR, D), lambda i: (i, 0)),
        out_shape=jax.ShapeDtypeStruct((N, D), jnp.float32),
    )(h, stats, gamma.reshape(1, D), beta.reshape(1, D))
    return out
